# Initial kernel scaffold; baseline (speedup 1.0000x reference)
import jax
import jax.numpy as jnp
from jax.experimental import pallas as pl


def kernel(input_ids, emb, lm_head):
    raise NotImplementedError("write your pallas kernel here")


# speedup vs baseline: 1.0232x; 1.0232x over previous
import jax
import jax.numpy as jnp
from jax.experimental import pallas as pl
from jax.experimental.pallas import tpu as pltpu

VOCAB = 32000
D_MODEL = 2048
S = 2048
MIN_TOKENS_RATIO = 0.25
THRESHOLD_SENSITIVITY = 0.5

_N_ROW = 256


def _mask_kernel(dcol_ref, drow_ref, conf_ref, out_ref):
    i = pl.program_id(0)
    dcol = dcol_ref[...]
    drow = drow_ref[...]
    jcol = jax.lax.broadcasted_iota(jnp.int32, (_N_ROW, 1), 0) + i * _N_ROW
    jrow = jax.lax.broadcasted_iota(jnp.int32, (1, S), 1)
    beats = (drow > dcol) | ((drow == dcol) & (jrow < jcol))
    rank = jnp.sum(beats.astype(jnp.int32), axis=1, keepdims=True)

    mean_conf = jnp.sum(conf_ref[...]) / jnp.float32(S)
    ratio = jnp.clip(
        jnp.float32(MIN_TOKENS_RATIO)
        * (1.0 + jnp.float32(THRESHOLD_SENSITIVITY) * (0.5 - mean_conf)),
        0.05, 1.0)
    k = jnp.maximum(1, (ratio * jnp.float32(S)).astype(jnp.int32))
    out_ref[...] = (rank < k).astype(jnp.int32)


def kernel(input_ids, emb, lm_head):
    x = jnp.take(emb, input_ids, axis=0)
    logits = x @ lm_head
    logp = jax.nn.log_softmax(logits, axis=-1)
    p = jnp.exp(logp)
    entropy = -jnp.sum(p * logp, axis=-1)
    confidence = jnp.max(p, axis=-1)
    ent_norm = entropy / jnp.log(float(VOCAB))
    difficulty = 0.5 * (1.0 - confidence) + 0.5 * ent_norm
    difficulty = difficulty.reshape(-1)
    confidence = confidence.reshape(-1)

    mask_i32 = pl.pallas_call(
        _mask_kernel,
        grid=(S // _N_ROW,),
        in_specs=[
            pl.BlockSpec((_N_ROW, 1), lambda i: (i, 0)),
            pl.BlockSpec((1, S), lambda i: (0, 0)),
            pl.BlockSpec((1, S), lambda i: (0, 0)),
        ],
        out_specs=pl.BlockSpec((_N_ROW, 1), lambda i: (i, 0)),
        out_shape=jax.ShapeDtypeStruct((S, 1), jnp.int32),
        compiler_params=pltpu.CompilerParams(
            dimension_semantics=("arbitrary",)),
    )(difficulty.reshape(S, 1), difficulty.reshape(1, S),
      confidence.reshape(1, S))
    mask = mask_i32.reshape(S).astype(bool)
    return difficulty, mask


# speedup vs baseline: 1.0261x; 1.0028x over previous
import jax
import jax.numpy as jnp
from jax.experimental import pallas as pl
from jax.experimental.pallas import tpu as pltpu

VOCAB = 32000
D_MODEL = 2048
S = 2048
MIN_TOKENS_RATIO = 0.25
THRESHOLD_SENSITIVITY = 0.5

_RB = 32
_N_ROW = 256
_NC = VOCAB // 128
_SEGS = (63, 63, 63, 61)


def _lane_reduce(acc):
    su = acc[:, 0:8]
    for t in range(1, 16):
        su = su + acc[:, t * 8:(t + 1) * 8]
    r = su[:, 0:4] + su[:, 4:8]
    r = r[:, 0:2] + r[:, 2:4]
    return r[:, 0:1] + r[:, 1:2]


def _reduce_sum(arr):
    total = None
    c0 = 0
    for n in _SEGS:
        acc = arr[:, c0 * 128:(c0 + 1) * 128]
        for c in range(c0 + 1, c0 + n):
            acc = acc + arr[:, c * 128:(c + 1) * 128]
        part = _lane_reduce(acc)
        total = part if total is None else total + part
        c0 += n
    return total


def _stats_kernel(l_ref, diff_ref, conf_ref):
    blk = l_ref[...]
    m = jnp.max(blk, axis=-1, keepdims=True)
    t = blk - m
    s = _reduce_sum(jnp.exp(t))
    logp = t - jnp.log(s)
    p = jnp.exp(logp)
    ent = -_reduce_sum(p * logp)
    conf = jnp.max(p, axis=-1, keepdims=True)
    ent_norm = ent / jnp.log(jnp.float32(float(VOCAB)))
    diff_ref[...] = 0.5 * (1.0 - conf) + 0.5 * ent_norm
    conf_ref[...] = conf


def _mask_kernel(dcol_ref, drow_ref, conf_ref, out_ref):
    i = pl.program_id(0)
    dcol = dcol_ref[...]
    drow = drow_ref[...]
    jcol = jax.lax.broadcasted_iota(jnp.int32, (_N_ROW, 1), 0) + i * _N_ROW
    jrow = jax.lax.broadcasted_iota(jnp.int32, (1, S), 1)
    beats = (drow > dcol) | ((drow == dcol) & (jrow < jcol))
    rank = jnp.sum(beats.astype(jnp.int32), axis=1, keepdims=True)

    mean_conf = jnp.sum(conf_ref[...]) / jnp.float32(S)
    ratio = jnp.clip(
        jnp.float32(MIN_TOKENS_RATIO)
        * (1.0 + jnp.float32(THRESHOLD_SENSITIVITY) * (0.5 - mean_conf)),
        0.05, 1.0)
    k = jnp.maximum(1, (ratio * jnp.float32(S)).astype(jnp.int32))
    out_ref[...] = (rank < k).astype(jnp.int32)


def kernel(input_ids, emb, lm_head):
    x = jnp.take(emb, input_ids, axis=0)
    logits = (x @ lm_head).reshape(S, VOCAB)

    diff_col, conf_col = pl.pallas_call(
        _stats_kernel,
        grid=(S // _RB,),
        in_specs=[pl.BlockSpec((_RB, VOCAB), lambda i: (i, 0))],
        out_specs=[pl.BlockSpec((_RB, 1), lambda i: (i, 0))] * 2,
        out_shape=[jax.ShapeDtypeStruct((S, 1), jnp.float32)] * 2,
        compiler_params=pltpu.CompilerParams(
            dimension_semantics=("arbitrary",)),
    )(logits)

    mask_i32 = pl.pallas_call(
        _mask_kernel,
        grid=(S // _N_ROW,),
        in_specs=[
            pl.BlockSpec((_N_ROW, 1), lambda i: (i, 0)),
            pl.BlockSpec((1, S), lambda i: (0, 0)),
            pl.BlockSpec((1, S), lambda i: (0, 0)),
        ],
        out_specs=pl.BlockSpec((_N_ROW, 1), lambda i: (i, 0)),
        out_shape=jax.ShapeDtypeStruct((S, 1), jnp.int32),
        compiler_params=pltpu.CompilerParams(
            dimension_semantics=("arbitrary",)),
    )(diff_col, diff_col.reshape(1, S), conf_col.reshape(1, S))

    difficulty = diff_col.reshape(-1)
    mask = mask_i32.reshape(S).astype(bool)
    return difficulty, mask
